# baseline (speedup 1.0000x reference)
import functools

import jax
import jax.numpy as jnp
from jax.experimental import pallas as pl
from jax.experimental.pallas import tpu as pltpu


def _gem_planes_kernel(x_ref, o_ref, *, hw, eps, inv_hw, inv_p):
    def body(i, acc):
        x = jnp.maximum(x_ref[i], jnp.float32(eps))
        return acc + x * x * x
    acc = jax.lax.fori_loop(
        0, hw, body, jnp.zeros(o_ref.shape, jnp.float32), unroll=True)
    o_ref[...] = jnp.power(acc * jnp.float32(inv_hw), jnp.float32(inv_p))


def _gem(x, p=3.0, eps=1e-6):
    N, C, H, W = x.shape
    HW = H * W
    xt = jnp.transpose(x, (2, 3, 0, 1)).reshape(HW, N, C)

    bn = 8
    while N % bn != 0:
        bn //= 2
    grid = N // bn

    kernel_fn = functools.partial(
        _gem_planes_kernel, hw=HW, eps=float(eps), inv_hw=1.0 / float(HW),
        inv_p=1.0 / float(p))
    out = pl.pallas_call(
        kernel_fn,
        out_shape=jax.ShapeDtypeStruct((N, C), x.dtype),
        grid=(grid,),
        in_specs=[pl.BlockSpec((HW, bn, C), lambda j: (0, j, 0))],
        out_specs=pl.BlockSpec((bn, C), lambda j: (j, 0)),
        compiler_params=pltpu.CompilerParams(
            dimension_semantics=("parallel",),
            vmem_limit_bytes=int(32 << 20)),
    )(xt)
    return out.reshape(N, C, 1, 1)


def kernel(x):
    return _gem(x, p=3.0, eps=1e-6)


# speedup vs baseline: 1.1166x; 1.1166x over previous
import functools

import jax
import jax.numpy as jnp
from jax.experimental import pallas as pl
from jax.experimental.pallas import tpu as pltpu


def _gem_planes_kernel(x_ref, o_ref, *, hw, eps, inv_hw, inv_p):
    def body(i, acc):
        x = jnp.maximum(x_ref[i], jnp.float32(eps))
        return acc + x * x * x
    acc = jax.lax.fori_loop(
        0, hw, body, jnp.zeros(o_ref.shape, jnp.float32), unroll=True)
    o_ref[...] = jnp.power(acc * jnp.float32(inv_hw), jnp.float32(inv_p))


def _gem(x, p=3.0, eps=1e-6):
    N, C, H, W = x.shape
    HW = H * W
    xt = jnp.transpose(x, (2, 3, 0, 1)).reshape(HW, N, C)

    bn = 16
    while N % bn != 0:
        bn //= 2
    grid = N // bn

    kernel_fn = functools.partial(
        _gem_planes_kernel, hw=HW, eps=float(eps), inv_hw=1.0 / float(HW),
        inv_p=1.0 / float(p))
    out = pl.pallas_call(
        kernel_fn,
        out_shape=jax.ShapeDtypeStruct((N, C), x.dtype),
        grid=(grid,),
        in_specs=[pl.BlockSpec((HW, bn, C), lambda j: (0, j, 0))],
        out_specs=pl.BlockSpec((bn, C), lambda j: (j, 0)),
        compiler_params=pltpu.CompilerParams(
            dimension_semantics=("parallel",),
            vmem_limit_bytes=int(32 << 20)),
    )(xt)
    return out.reshape(N, C, 1, 1)


def kernel(x):
    return _gem(x, p=3.0, eps=1e-6)


# speedup vs baseline: 1.1697x; 1.0476x over previous
import functools

import jax
import jax.numpy as jnp
from jax.experimental import pallas as pl
from jax.experimental.pallas import tpu as pltpu


def _gem_planes_kernel(x_ref, o_ref, *, hw, eps, inv_hw, inv_p):
    def body(i, acc):
        x = jnp.maximum(x_ref[i], jnp.float32(eps))
        return acc + x * x * x
    acc = jax.lax.fori_loop(
        0, hw, body, jnp.zeros(x_ref.shape[1:], jnp.float32), unroll=True)
    res = jnp.power(acc * jnp.float32(inv_hw), jnp.float32(inv_p))
    o_ref[...] = res.reshape(o_ref.shape)


def _gem(x, p=3.0, eps=1e-6):
    N, C, H, W = x.shape
    HW = H * W
    xt = jnp.transpose(x, (2, 3, 0, 1)).reshape(HW, N, C)

    bn = 16
    while N % bn != 0:
        bn //= 2
    grid = N // bn

    kernel_fn = functools.partial(
        _gem_planes_kernel, hw=HW, eps=float(eps), inv_hw=1.0 / float(HW),
        inv_p=1.0 / float(p))
    out = pl.pallas_call(
        kernel_fn,
        out_shape=jax.ShapeDtypeStruct((N * C,), x.dtype),
        grid=(grid,),
        in_specs=[pl.BlockSpec((HW, bn, C), lambda j: (0, j, 0))],
        out_specs=pl.BlockSpec((bn * C,), lambda j: (j,)),
        compiler_params=pltpu.CompilerParams(
            dimension_semantics=("parallel",),
            vmem_limit_bytes=int(32 << 20)),
    )(xt)
    return out.reshape(N, C, 1, 1)


def kernel(x):
    return _gem(x, p=3.0, eps=1e-6)
